# Initial kernel scaffold; baseline (speedup 1.0000x reference)
import jax
import jax.numpy as jnp
from jax.experimental import pallas as pl


def kernel(x, edge_index, W_in, b_in, W_t1, b_t1, W_t2, b_t2, W_ih, b_ih, W_hh, b_hh, W_out, b_out):
    raise NotImplementedError("write your pallas kernel here")


# speedup vs baseline: 5.6374x; 5.6374x over previous
import functools

import jax
import jax.numpy as jnp
from jax import lax
from jax.experimental import pallas as pl
from jax.experimental.pallas import tpu as pltpu
from jax.experimental.pallas import tpu_sc as plsc

_NC = 2
_NS = 16
_NW = _NC * _NS
_LANES = 16

_ROUNDS = 5
_BLK = 512


def _pre_body(x_ref, w_in_ref, b_in_ref, w_t1_ref, b_t1_ref, w_t2_ref,
              b_t2_ref, h_ref, tau_ref):
    x = x_ref[...]
    h = jnp.maximum(
        lax.dot_general(x, w_in_ref[...], (((1,), (1,)), ((), ())),
                        preferred_element_type=jnp.float32) + b_in_ref[...], 0.0)
    h_ref[...] = h
    t1 = jnp.maximum(
        lax.dot_general(x, w_t1_ref[...], (((1,), (1,)), ((), ())),
                        preferred_element_type=jnp.float32) + b_t1_ref[...], 0.0)
    t2 = jax.nn.sigmoid(
        lax.dot_general(t1, w_t2_ref[...], (((1,), (1,)), ((), ())),
                        preferred_element_type=jnp.float32) + b_t2_ref[...])
    tau_ref[...] = jnp.mean(t2, axis=1, keepdims=True)


def _gru_body(p0_ref, p1_ref, tau_ref, h_ref, w_ih_ref, b_ih_ref, w_hh_ref,
              b_hh_ref, out_ref):
    agg = (p0_ref[...] + p1_ref[...]) * tau_ref[...]
    h = h_ref[...]
    gi = lax.dot_general(agg, w_ih_ref[...], (((1,), (1,)), ((), ())),
                         preferred_element_type=jnp.float32) + b_ih_ref[...]
    gh = lax.dot_general(h, w_hh_ref[...], (((1,), (1,)), ((), ())),
                         preferred_element_type=jnp.float32) + b_hh_ref[...]
    hdim = h.shape[1]
    r = jax.nn.sigmoid(gi[:, :hdim] + gh[:, :hdim])
    z = jax.nn.sigmoid(gi[:, hdim:2 * hdim] + gh[:, hdim:2 * hdim])
    n = jnp.tanh(gi[:, 2 * hdim:] + r * gh[:, 2 * hdim:])
    out_ref[...] = (1.0 - z) * n + z * h


def _gru_out_body(p0_ref, p1_ref, tau_ref, h_ref, w_ih_ref, b_ih_ref,
                  w_hh_ref, b_hh_ref, w_out_ref, b_out_ref, out_ref):
    agg = (p0_ref[...] + p1_ref[...]) * tau_ref[...]
    h = h_ref[...]
    gi = lax.dot_general(agg, w_ih_ref[...], (((1,), (1,)), ((), ())),
                         preferred_element_type=jnp.float32) + b_ih_ref[...]
    gh = lax.dot_general(h, w_hh_ref[...], (((1,), (1,)), ((), ())),
                         preferred_element_type=jnp.float32) + b_hh_ref[...]
    hdim = h.shape[1]
    r = jax.nn.sigmoid(gi[:, :hdim] + gh[:, :hdim])
    z = jax.nn.sigmoid(gi[:, hdim:2 * hdim] + gh[:, hdim:2 * hdim])
    n = jnp.tanh(gi[:, 2 * hdim:] + r * gh[:, 2 * hdim:])
    hn = (1.0 - z) * n + z * h
    out_ref[...] = lax.dot_general(
        hn, w_out_ref[...], (((1,), (1,)), ((), ())),
        preferred_element_type=jnp.float32) + b_out_ref[...]


def _row_spec(d):
    return pl.BlockSpec((_BLK, d), lambda i: (i, 0))


def _full_spec(shape):
    nd = len(shape)
    return pl.BlockSpec(shape, lambda i: (0,) * nd)


def _make_sc_agg(n, e, d):
    ew = e // _NW
    chunk = 80
    assert ew % chunk == 0 and (ew * _NW) == e
    nchunks = ew // chunk
    npad = ((n + 8 * _NS - 1) // (8 * _NS)) * (8 * _NS)
    rows_per_sub = npad // _NS

    mesh = plsc.VectorSubcoreMesh(core_axis_name="c", subcore_axis_name="s",
                                  num_cores=_NC, num_subcores=_NS)

    @functools.partial(
        pl.kernel,
        out_type=(jax.ShapeDtypeStruct((npad, d), jnp.float32),
                  jax.ShapeDtypeStruct((npad, d), jnp.float32)),
        mesh=mesh,
        scratch_types=dict(
            rowi=pltpu.VMEM((chunk,), jnp.int32),
            coli=pltpu.VMEM((chunk,), jnp.int32),
            buf_a=pltpu.VMEM((chunk, d), jnp.float32),
            buf_b=pltpu.VMEM((chunk, d), jnp.float32),
            buf_c=pltpu.VMEM((chunk, d), jnp.float32),
            acc=pltpu.MemorySpace.VMEM_SHARED((npad, d), jnp.float32),
            sem_a=pltpu.SemaphoreType.DMA,
            sem_b=pltpu.SemaphoreType.DMA,
        ),
    )
    def sc_agg(h_hbm, rows_hbm, cols_hbm, zeros_hbm, out0, out1, *, rowi,
               coli, buf_a, buf_b, buf_c, acc, sem_a, sem_b):
        c = lax.axis_index("c")
        s = lax.axis_index("s")
        wid = s * _NC + c

        zslice = pl.ds(s * rows_per_sub, rows_per_sub)
        pltpu.sync_copy(zeros_hbm.at[zslice], acc.at[zslice])
        plsc.subcore_barrier()

        base_w = wid * ew

        @pl.loop(0, nchunks)
        def _chunk_loop(k):
            base = base_w + k * chunk
            pltpu.sync_copy(rows_hbm.at[pl.ds(base, chunk)], rowi)
            pltpu.sync_copy(cols_hbm.at[pl.ds(base, chunk)], coli)
            cp_a = pltpu.async_copy(h_hbm.at[rowi], buf_a, sem_a)
            cp_b = pltpu.async_copy(h_hbm.at[coli], buf_b, sem_b)
            cp_a.wait()
            cp_b.wait()

            @plsc.parallel_loop(0, chunk, unroll=2)
            def _edge_loop(ei):
                for j in range(d // _LANES):
                    sl = pl.ds(j * _LANES, _LANES)
                    buf_c[ei, sl] = jnp.abs(buf_a[ei, sl] - buf_b[ei, sl])

            pltpu.sync_copy(buf_c, acc.at[rowi], add=True)

        plsc.subcore_barrier()
        oslice = pl.ds(s * rows_per_sub, rows_per_sub)

        @pl.when(c == 0)
        def _():
            pltpu.sync_copy(acc.at[oslice], out0.at[oslice])

        @pl.when(c == 1)
        def _():
            pltpu.sync_copy(acc.at[oslice], out1.at[oslice])

    return sc_agg


def kernel(x, edge_index, W_in, b_in, W_t1, b_t1, W_t2, b_t2, W_ih, b_ih,
           W_hh, b_hh, W_out, b_out):
    n, d = x.shape
    h_dim = W_in.shape[0]
    o_dim = W_out.shape[0]
    e = edge_index.shape[1]
    grid = (pl.cdiv(n, _BLK),)

    rows = edge_index[0]
    cols = edge_index[1]
    npad = ((n + 8 * _NS - 1) // (8 * _NS)) * (8 * _NS)
    zeros = jnp.zeros((npad, h_dim), jnp.float32)

    b_in2 = b_in.reshape(1, -1)
    b_t12 = b_t1.reshape(1, -1)
    b_t22 = b_t2.reshape(1, -1)
    b_ih2 = b_ih.reshape(1, -1)
    b_hh2 = b_hh.reshape(1, -1)
    b_out2 = b_out.reshape(1, -1)

    h, tau = pl.pallas_call(
        _pre_body,
        grid=grid,
        in_specs=[_row_spec(d), _full_spec(W_in.shape), _full_spec((1, h_dim)),
                  _full_spec(W_t1.shape), _full_spec((1, h_dim)),
                  _full_spec(W_t2.shape), _full_spec((1, W_t2.shape[0]))],
        out_specs=[_row_spec(h_dim), pl.BlockSpec((_BLK, 1), lambda i: (i, 0))],
        out_shape=[jax.ShapeDtypeStruct((n, h_dim), jnp.float32),
                   jax.ShapeDtypeStruct((n, 1), jnp.float32)],
    )(x, W_in, b_in2, W_t1, b_t12, W_t2, b_t22)

    sc_agg = _make_sc_agg(n, e, h_dim)

    gru_in_specs = [_row_spec(h_dim), _row_spec(h_dim),
                    pl.BlockSpec((_BLK, 1), lambda i: (i, 0)), _row_spec(h_dim),
                    _full_spec(W_ih.shape), _full_spec((1, 3 * h_dim)),
                    _full_spec(W_hh.shape), _full_spec((1, 3 * h_dim))]

    for r in range(_ROUNDS):
        p0, p1 = sc_agg(h, rows, cols, zeros)
        if r < _ROUNDS - 1:
            h = pl.pallas_call(
                _gru_body,
                grid=grid,
                in_specs=gru_in_specs,
                out_specs=_row_spec(h_dim),
                out_shape=jax.ShapeDtypeStruct((n, h_dim), jnp.float32),
            )(p0, p1, tau, h, W_ih, b_ih2, W_hh, b_hh2)
        else:
            out = pl.pallas_call(
                _gru_out_body,
                grid=grid,
                in_specs=gru_in_specs + [_full_spec(W_out.shape),
                                         _full_spec((1, o_dim))],
                out_specs=_row_spec(o_dim),
                out_shape=jax.ShapeDtypeStruct((n, o_dim), jnp.float32),
            )(p0, p1, tau, h, W_ih, b_ih2, W_hh, b_hh2, W_out, b_out2)
    return out


# speedup vs baseline: 10.8262x; 1.9204x over previous
import functools

import jax
import jax.numpy as jnp
from jax import lax
from jax.experimental import pallas as pl
from jax.experimental.pallas import tpu as pltpu
from jax.experimental.pallas import tpu_sc as plsc

_NC = 2
_NS = 16
_NW = _NC * _NS
_LANES = 16

_ROUNDS = 5
_BLK = 512


def _pre_body(x_ref, w_in_ref, b_in_ref, w_t1_ref, b_t1_ref, w_t2_ref,
              b_t2_ref, h_ref, tau_ref):
    x = x_ref[...]
    h = jnp.maximum(
        lax.dot_general(x, w_in_ref[...], (((1,), (1,)), ((), ())),
                        preferred_element_type=jnp.float32) + b_in_ref[...], 0.0)
    h_ref[...] = h
    t1 = jnp.maximum(
        lax.dot_general(x, w_t1_ref[...], (((1,), (1,)), ((), ())),
                        preferred_element_type=jnp.float32) + b_t1_ref[...], 0.0)
    t2 = jax.nn.sigmoid(
        lax.dot_general(t1, w_t2_ref[...], (((1,), (1,)), ((), ())),
                        preferred_element_type=jnp.float32) + b_t2_ref[...])
    tau_ref[...] = jnp.mean(t2, axis=1, keepdims=True)


def _gru_body(p0_ref, p1_ref, tau_ref, h_ref, w_ih_ref, b_ih_ref, w_hh_ref,
              b_hh_ref, out_ref):
    agg = (p0_ref[...] + p1_ref[...]) * tau_ref[...]
    h = h_ref[...]
    gi = lax.dot_general(agg, w_ih_ref[...], (((1,), (1,)), ((), ())),
                         preferred_element_type=jnp.float32) + b_ih_ref[...]
    gh = lax.dot_general(h, w_hh_ref[...], (((1,), (1,)), ((), ())),
                         preferred_element_type=jnp.float32) + b_hh_ref[...]
    hdim = h.shape[1]
    r = jax.nn.sigmoid(gi[:, :hdim] + gh[:, :hdim])
    z = jax.nn.sigmoid(gi[:, hdim:2 * hdim] + gh[:, hdim:2 * hdim])
    n = jnp.tanh(gi[:, 2 * hdim:] + r * gh[:, 2 * hdim:])
    out_ref[...] = (1.0 - z) * n + z * h


def _gru_out_body(p0_ref, p1_ref, tau_ref, h_ref, w_ih_ref, b_ih_ref,
                  w_hh_ref, b_hh_ref, w_out_ref, b_out_ref, out_ref):
    agg = (p0_ref[...] + p1_ref[...]) * tau_ref[...]
    h = h_ref[...]
    gi = lax.dot_general(agg, w_ih_ref[...], (((1,), (1,)), ((), ())),
                         preferred_element_type=jnp.float32) + b_ih_ref[...]
    gh = lax.dot_general(h, w_hh_ref[...], (((1,), (1,)), ((), ())),
                         preferred_element_type=jnp.float32) + b_hh_ref[...]
    hdim = h.shape[1]
    r = jax.nn.sigmoid(gi[:, :hdim] + gh[:, :hdim])
    z = jax.nn.sigmoid(gi[:, hdim:2 * hdim] + gh[:, hdim:2 * hdim])
    n = jnp.tanh(gi[:, 2 * hdim:] + r * gh[:, 2 * hdim:])
    hn = (1.0 - z) * n + z * h
    out_ref[...] = lax.dot_general(
        hn, w_out_ref[...], (((1,), (1,)), ((), ())),
        preferred_element_type=jnp.float32) + b_out_ref[...]


def _row_spec(d):
    return pl.BlockSpec((_BLK, d), lambda i: (i, 0))


def _full_spec(shape):
    nd = len(shape)
    return pl.BlockSpec(shape, lambda i: (0,) * nd)


def _make_sc_agg(n, e, d):
    ew = e // _NW
    chunk = 40
    nb = 2
    seg = 50
    assert ew % (chunk * seg) == 0 and (ew * _NW) == e
    nseg = ew // (chunk * seg)
    assert seg % nb == 0
    npad = ((n + 8 * _NS - 1) // (8 * _NS)) * (8 * _NS)
    rows_per_sub = npad // _NS

    mesh = plsc.VectorSubcoreMesh(core_axis_name="c", subcore_axis_name="s",
                                  num_cores=_NC, num_subcores=_NS)

    @functools.partial(
        pl.kernel,
        out_type=(jax.ShapeDtypeStruct((npad, d), jnp.float32),
                  jax.ShapeDtypeStruct((npad, d), jnp.float32)),
        mesh=mesh,
        scratch_types=dict(
            idx_r=pltpu.VMEM((seg, chunk), jnp.int32),
            idx_c=pltpu.VMEM((seg, chunk), jnp.int32),
            buf_a=pltpu.VMEM((nb, chunk, d), jnp.float32),
            buf_b=pltpu.VMEM((nb, chunk, d), jnp.float32),
            buf_c=pltpu.VMEM((nb, chunk, d), jnp.float32),
            acc=pltpu.MemorySpace.VMEM_SHARED((npad, d), jnp.float32),
            sem_a=pltpu.SemaphoreType.DMA((nb,)),
            sem_b=pltpu.SemaphoreType.DMA((nb,)),
            sem_s=pltpu.SemaphoreType.DMA((nb,)),
        ),
    )
    def sc_agg(h_hbm, rows_hbm, cols_hbm, zeros_hbm, out0, out1, *, idx_r,
               idx_c, buf_a, buf_b, buf_c, acc, sem_a, sem_b, sem_s):
        c = lax.axis_index("c")
        s = lax.axis_index("s")
        wid = s * _NC + c

        zslice = pl.ds(s * rows_per_sub, rows_per_sub)
        pltpu.sync_copy(zeros_hbm.at[zslice], acc.at[zslice])
        plsc.subcore_barrier()

        def gathers(k, b):
            ga = pltpu.make_async_copy(h_hbm.at[idx_r.at[k]], buf_a.at[b],
                                       sem_a.at[b])
            gb = pltpu.make_async_copy(h_hbm.at[idx_c.at[k]], buf_b.at[b],
                                       sem_b.at[b])
            return ga, gb

        def scatter(k, b):
            return pltpu.make_async_copy(buf_c.at[b], acc.at[idx_r.at[k]],
                                         sem_s.at[b])

        @pl.loop(0, nseg)
        def _seg_loop(sg):
            @pl.when(sg > 0)
            def _():
                for b in range(nb):
                    scatter(seg - nb + b, b).wait()

            pltpu.sync_copy(rows_hbm.at[wid, sg], idx_r)
            pltpu.sync_copy(cols_hbm.at[wid, sg], idx_c)

            for b in range(nb):
                ga, gb = gathers(b, b)
                ga.start()
                gb.start()

            @pl.loop(0, seg, step=nb)
            def _chunk_loop(g):
                for b in range(nb):
                    k = g + b
                    ga, gb = gathers(k, b)
                    ga.wait()
                    gb.wait()

                    @pl.when(k >= nb)
                    def _():
                        scatter(k, b).wait()

                    @plsc.parallel_loop(0, chunk, unroll=4)
                    def _edge_loop(ei):
                        for j in range(d // _LANES):
                            sl = pl.ds(j * _LANES, _LANES)
                            buf_c[b, ei, sl] = jnp.abs(
                                buf_a[b, ei, sl] - buf_b[b, ei, sl])

                    scatter(k, b).start(add=True)

                    @pl.when(k + nb < seg)
                    def _():
                        ga2, gb2 = gathers(k + nb, b)
                        ga2.start()
                        gb2.start()

        for b in range(nb):
            scatter(seg - nb + b, b).wait()

        plsc.subcore_barrier()
        oslice = pl.ds(s * rows_per_sub, rows_per_sub)

        @pl.when(c == 0)
        def _():
            pltpu.sync_copy(acc.at[oslice], out0.at[oslice])

        @pl.when(c == 1)
        def _():
            pltpu.sync_copy(acc.at[oslice], out1.at[oslice])

    return sc_agg


def kernel(x, edge_index, W_in, b_in, W_t1, b_t1, W_t2, b_t2, W_ih, b_ih,
           W_hh, b_hh, W_out, b_out):
    n, d = x.shape
    h_dim = W_in.shape[0]
    o_dim = W_out.shape[0]
    e = edge_index.shape[1]
    grid = (pl.cdiv(n, _BLK),)

    chunk = 40
    seg = 50
    nseg = (e // _NW) // (chunk * seg)
    rows = edge_index[0].reshape(_NW, nseg, seg, chunk)
    cols = edge_index[1].reshape(_NW, nseg, seg, chunk)
    npad = ((n + 8 * _NS - 1) // (8 * _NS)) * (8 * _NS)
    zeros = jnp.zeros((npad, h_dim), jnp.float32)

    b_in2 = b_in.reshape(1, -1)
    b_t12 = b_t1.reshape(1, -1)
    b_t22 = b_t2.reshape(1, -1)
    b_ih2 = b_ih.reshape(1, -1)
    b_hh2 = b_hh.reshape(1, -1)
    b_out2 = b_out.reshape(1, -1)

    h, tau = pl.pallas_call(
        _pre_body,
        grid=grid,
        in_specs=[_row_spec(d), _full_spec(W_in.shape), _full_spec((1, h_dim)),
                  _full_spec(W_t1.shape), _full_spec((1, h_dim)),
                  _full_spec(W_t2.shape), _full_spec((1, W_t2.shape[0]))],
        out_specs=[_row_spec(h_dim), pl.BlockSpec((_BLK, 1), lambda i: (i, 0))],
        out_shape=[jax.ShapeDtypeStruct((n, h_dim), jnp.float32),
                   jax.ShapeDtypeStruct((n, 1), jnp.float32)],
    )(x, W_in, b_in2, W_t1, b_t12, W_t2, b_t22)

    sc_agg = _make_sc_agg(n, e, h_dim)

    gru_in_specs = [_row_spec(h_dim), _row_spec(h_dim),
                    pl.BlockSpec((_BLK, 1), lambda i: (i, 0)), _row_spec(h_dim),
                    _full_spec(W_ih.shape), _full_spec((1, 3 * h_dim)),
                    _full_spec(W_hh.shape), _full_spec((1, 3 * h_dim))]

    for r in range(_ROUNDS):
        p0, p1 = sc_agg(h, rows, cols, zeros)
        if r < _ROUNDS - 1:
            h = pl.pallas_call(
                _gru_body,
                grid=grid,
                in_specs=gru_in_specs,
                out_specs=_row_spec(h_dim),
                out_shape=jax.ShapeDtypeStruct((n, h_dim), jnp.float32),
            )(p0, p1, tau, h, W_ih, b_ih2, W_hh, b_hh2)
        else:
            out = pl.pallas_call(
                _gru_out_body,
                grid=grid,
                in_specs=gru_in_specs + [_full_spec(W_out.shape),
                                         _full_spec((1, o_dim))],
                out_specs=_row_spec(o_dim),
                out_shape=jax.ShapeDtypeStruct((n, o_dim), jnp.float32),
            )(p0, p1, tau, h, W_ih, b_ih2, W_hh, b_hh2, W_out, b_out2)
    return out
